# baseline (speedup 1.0000x reference)
import functools

import jax
import jax.numpy as jnp
import numpy as np
from jax import lax
from jax.experimental import pallas as pl
from jax.experimental.pallas import tpu as pltpu
from jax.experimental.pallas import tpu_sc as plsc

EPS = 1e-4
ALPHA = 4.0
N = 50000
NP = 800000
PS = 3
D = 128
H = 32

NW = 32
CH = 128
CHUNKS = 196
PPW = CH * CHUNKS
NPP = NW * PPW
N_POOL = 50048
RPT = N_POOL // 16
BN = 5000
NB = N // BN

PERM = np.empty(H, dtype=np.int32)
PERM[0::2] = np.arange(16)
PERM[1::2] = np.arange(16) + 16


def _tc1_body(f_ref, w_ref, e_ref, lt_ref):
    w2 = w_ref[...].reshape(PS * D, H)
    colnorm = jnp.maximum(jnp.sqrt(jnp.sum(w2 * w2, axis=0, keepdims=True)), EPS)
    wn = w2 / colnorm
    f = f_ref[...]
    inv = lax.rsqrt(jnp.maximum(jnp.sum(f * f, axis=1, keepdims=True), EPS * EPS))
    fn = f * inv
    wj = wn.reshape(PS, D, H)
    for j in range(PS):
        e_ref[j] = (jnp.dot(fn, wj[j], preferred_element_type=jnp.float32)
                    * (1.0 / PS)).astype(jnp.bfloat16)

    @pl.when(pl.program_id(0) == 0)
    def _():
        gram = jnp.exp(
            ALPHA * (lax.dot_general(wn, wn, (((0,), (0,)), ((), ())),
                                     preferred_element_type=jnp.float32) - 1.0))
        c = jnp.sqrt(jnp.sum(gram * gram))
        eye = jnp.eye(H, dtype=jnp.float32)
        y = gram * (1.0 / c)
        z = eye
        for _ in range(24):
            t = 1.5 * eye - 0.5 * jnp.dot(z, y, preferred_element_type=jnp.float32)
            y = jnp.dot(y, t, preferred_element_type=jnp.float32)
            z = jnp.dot(t, z, preferred_element_type=jnp.float32)
        kk = lax.broadcasted_iota(jnp.int32, (H, H), 1)
        m = (lax.broadcasted_iota(jnp.int32, (H, H), 0)
             == kk // 2 + 16 * (kk % 2)).astype(jnp.float32)
        lt = jnp.dot(m, z * lax.rsqrt(c), preferred_element_type=jnp.float32)
        lt_ref[...] = lax.dot_general(lt, m, (((1,), (1,)), ((), ())),
                                      preferred_element_type=jnp.float32)


_tc1 = pl.pallas_call(
    _tc1_body,
    grid=(NB,),
    in_specs=[
        pl.BlockSpec((BN, D), lambda i: (i, 0)),
        pl.BlockSpec((PS, D, H), lambda i: (0, 0, 0)),
    ],
    out_specs=[
        pl.BlockSpec((PS, BN, H), lambda i: (0, i, 0)),
        pl.BlockSpec((H, H), lambda i: (0, 0)),
    ],
    out_shape=[
        jax.ShapeDtypeStruct((PS, N, H), jnp.bfloat16),
        jax.ShapeDtypeStruct((H, H), jnp.float32),
    ],
)


def _sc_body(e_hbm, g4_hbm, out_hbm,
             b0, b1, r0, r1, o0, o1, pooled, sem_i, sg0, sg1):
    c = lax.axis_index("c")
    s = lax.axis_index("s")
    wid = s * 2 + c
    cbase = wid * CHUNKS

    def zero_row(p, _):
        for h in (0, 16):
            o0[p, pl.ds(h, 16)] = jnp.zeros((16,), jnp.float32)
        return 0

    lax.fori_loop(0, CH, zero_row, 0)

    nfull = RPT // CH
    rem = RPT - nfull * CH

    def zero_pool(i, _):
        pltpu.async_copy(o0, pooled.at[pl.ds(s * RPT + i * CH, CH)], sg1)
        return 0

    lax.fori_loop(0, nfull, zero_pool, 0)
    pltpu.async_copy(o0.at[pl.ds(0, rem)],
                     pooled.at[pl.ds(s * RPT + nfull * CH, rem)], sg0)

    def zero_wait(i, _):
        pltpu.make_async_copy(o0, pooled.at[pl.ds(s * RPT, CH)], sg1).wait()
        return 0

    lax.fori_loop(0, nfull, zero_wait, 0)
    pltpu.make_async_copy(o0.at[pl.ds(0, rem)],
                          pooled.at[pl.ds(s * RPT, rem)], sg0).wait()
    plsc.subcore_barrier()

    def fire_gathers(b, r, sg):
        for j in range(PS):
            pltpu.async_copy(e_hbm.at[b.at[j]], r.at[j], sg)

    def drain_gathers(r, sg):
        for j in range(PS):
            pltpu.make_async_copy(e_hbm.at[pl.ds(0, CH)], r.at[j], sg).wait()

    def compute(r, o):
        def body(pb, _):
            for u in range(4):
                p = pb * 4 + u
                v = r[0, p, :] + r[1, p, :] + r[2, p, :]
                va, vb = plsc.unpack(v, format=plsc.PackFormat.INTERLEAVED,
                                     preferred_element_type=jnp.float32)
                o[p, pl.ds(0, 16)] = jnp.exp(ALPHA * va - ALPHA)
                o[p, pl.ds(16, 16)] = jnp.exp(ALPHA * vb - ALPHA)
            return 0

        lax.fori_loop(0, CH // 4, body, 0)

    pltpu.sync_copy(g4_hbm.at[cbase], b0)
    fire_gathers(b0, r0, sg0)
    pltpu.async_copy(g4_hbm.at[cbase + 1], b1, sem_i)

    def step(i, _):
        for u, bc, rc, oc, sgc, bn, rn, sgn in (
                (0, b0, r0, o0, sg0, b1, r1, sg1),
                (1, b1, r1, o1, sg1, b0, r0, sg0)):
            m = 2 * i + u
            drain_gathers(rc, sgc)

            @pl.when(m + 1 < CHUNKS)
            def _():
                pltpu.make_async_copy(g4_hbm.at[cbase], bn, sem_i).wait()
                fire_gathers(bn, rn, sgn)

            compute(rc, oc)
            pltpu.sync_copy(oc, pooled.at[bc.at[PS]], add=True)

            @pl.when(m + 2 < CHUNKS)
            def _():
                pltpu.async_copy(g4_hbm.at[cbase + m + 2], bc, sem_i)

        return 0

    lax.fori_loop(0, CHUNKS // 2, step, 0)
    plsc.subcore_barrier()
    pltpu.sync_copy(pooled.at[pl.ds(s * RPT, RPT)],
                    out_hbm.at[c, pl.ds(s * RPT, RPT)])


_sc_pool = functools.partial(
    pl.kernel,
    mesh=plsc.VectorSubcoreMesh(core_axis_name="c", subcore_axis_name="s"),
    compiler_params=pltpu.CompilerParams(use_tc_tiling_on_sc=False,
                                         needs_layout_passes=False),
    out_type=jax.ShapeDtypeStruct((2, N_POOL, H), jnp.float32),
    scratch_types=[
        pltpu.VMEM((PS + 1, CH), jnp.int32),
        pltpu.VMEM((PS + 1, CH), jnp.int32),
        pltpu.VMEM((PS, CH, H), jnp.bfloat16),
        pltpu.VMEM((PS, CH, H), jnp.bfloat16),
        pltpu.VMEM((CH, H), jnp.float32),
        pltpu.VMEM((CH, H), jnp.float32),
        pltpu.VMEM_SHARED((N_POOL, H), jnp.float32),
        pltpu.SemaphoreType.DMA,
        pltpu.SemaphoreType.DMA,
        pltpu.SemaphoreType.DMA,
    ],
)(_sc_body)


def _tc2_body(p_ref, ks_ref, lt_ref, o_ref):
    p = p_ref[0] + p_ref[1]
    cnt = jnp.maximum(ks_ref[...], 1).astype(jnp.float32)
    o_ref[...] = jnp.dot(p / cnt, lt_ref[...], preferred_element_type=jnp.float32)


_tc2 = pl.pallas_call(
    _tc2_body,
    grid=(NB,),
    in_specs=[
        pl.BlockSpec((2, BN, H), lambda i: (0, i, 0)),
        pl.BlockSpec((BN, 1), lambda i: (i, 0)),
        pl.BlockSpec((H, H), lambda i: (0, 0)),
    ],
    out_specs=pl.BlockSpec((BN, H), lambda i: (i, 0)),
    out_shape=jax.ShapeDtypeStruct((N, H), jnp.float32),
)


@jax.jit
def kernel(features, paths_indices, kernel_size, weight):
    e3, lintrans = _tc1(features, weight[:, :, jnp.asarray(PERM)])
    e_flat = e3.reshape(PS * N, H)

    cum = jnp.cumsum(kernel_size)
    z = jnp.zeros((NP,), jnp.int32).at[cum].add(1, mode='drop')
    seg = jnp.cumsum(z)
    segp = jnp.concatenate([seg, jnp.full((NPP - NP,), N, jnp.int32)])
    pad = jnp.zeros((NPP - NP,), jnp.int32)
    gs = [jnp.concatenate([paths_indices[:, j] + j * N, pad]) for j in range(PS)]
    g4 = jnp.stack(gs + [segp], axis=0).reshape(PS + 1, NPP // CH, CH)
    g4 = g4.transpose(1, 0, 2)

    pooled2 = _sc_pool(e_flat, g4)
    return _tc2(pooled2, kernel_size.reshape(N, 1), lintrans)


# speedup vs baseline: 1.0039x; 1.0039x over previous
import functools

import jax
import jax.numpy as jnp
from jax import lax
from jax.experimental import pallas as pl
from jax.experimental.pallas import tpu as pltpu
from jax.experimental.pallas import tpu_sc as plsc

EPS = 1e-4
ALPHA = 4.0
N = 50000
NP = 800000
PS = 3
D = 128
H = 32

NW = 32
CH = 128
CHUNKS = 198
PPW = CH * CHUNKS
NPP = NW * PPW
N_POOL = 50048
RPT = N_POOL // 16
BN = 5000
NB = N // BN


def _tc1_body(f_ref, w_ref, e_ref, lt_ref):
    w2 = w_ref[...].reshape(PS * D, H)
    colnorm = jnp.maximum(jnp.sqrt(jnp.sum(w2 * w2, axis=0, keepdims=True)), EPS)
    wn = w2 / colnorm
    f = f_ref[...]
    inv = lax.rsqrt(jnp.maximum(jnp.sum(f * f, axis=1, keepdims=True), EPS * EPS))
    fn = f * inv
    wj = wn.reshape(PS, D, H)
    for j in range(PS):
        e_ref[j] = jnp.dot(fn, wj[j], preferred_element_type=jnp.float32) * (1.0 / PS)

    @pl.when(pl.program_id(0) == 0)
    def _():
        gram = jnp.exp(
            ALPHA * (lax.dot_general(wn, wn, (((0,), (0,)), ((), ())),
                                     preferred_element_type=jnp.float32) - 1.0))
        c = jnp.sqrt(jnp.sum(gram * gram))
        eye = jnp.eye(H, dtype=jnp.float32)
        y = gram * (1.0 / c)
        z = eye
        for _ in range(24):
            t = 1.5 * eye - 0.5 * jnp.dot(z, y, preferred_element_type=jnp.float32)
            y = jnp.dot(y, t, preferred_element_type=jnp.float32)
            z = jnp.dot(t, z, preferred_element_type=jnp.float32)
        lt_ref[...] = z * lax.rsqrt(c)


_tc1 = pl.pallas_call(
    _tc1_body,
    grid=(NB,),
    in_specs=[
        pl.BlockSpec((BN, D), lambda i: (i, 0)),
        pl.BlockSpec((PS, D, H), lambda i: (0, 0, 0)),
    ],
    out_specs=[
        pl.BlockSpec((PS, BN, H), lambda i: (0, i, 0)),
        pl.BlockSpec((H, H), lambda i: (0, 0)),
    ],
    out_shape=[
        jax.ShapeDtypeStruct((PS, N, H), jnp.float32),
        jax.ShapeDtypeStruct((H, H), jnp.float32),
    ],
)


def _sc_body(e_hbm, g4_hbm, out_hbm,
             b0, b1, b2, r0, r1, pooled, sg0, sg1, si0, si1, si2, ss0, ss1):
    c = lax.axis_index("c")
    s = lax.axis_index("s")
    wid = s * 2 + c
    cbase = wid * CHUNKS

    def zero_row(p, _):
        for h in (0, 16):
            r0[0, p, pl.ds(h, 16)] = jnp.zeros((16,), jnp.float32)
        return 0

    lax.fori_loop(0, CH, zero_row, 0)

    nfull = RPT // CH
    rem = RPT - nfull * CH

    def zero_pool(i, _):
        pltpu.async_copy(r0.at[0], pooled.at[pl.ds(s * RPT + i * CH, CH)], sg1)
        return 0

    lax.fori_loop(0, nfull, zero_pool, 0)
    pltpu.async_copy(r0.at[0, pl.ds(0, rem)],
                     pooled.at[pl.ds(s * RPT + nfull * CH, rem)], sg0)

    def zero_wait(i, _):
        pltpu.make_async_copy(r0.at[0], pooled.at[pl.ds(s * RPT, CH)], sg1).wait()
        return 0

    lax.fori_loop(0, nfull, zero_wait, 0)
    pltpu.make_async_copy(r0.at[0, pl.ds(0, rem)],
                          pooled.at[pl.ds(s * RPT, rem)], sg0).wait()
    plsc.subcore_barrier()

    def fire_gathers(b, r, sg):
        for j in range(PS):
            pltpu.async_copy(e_hbm.at[b.at[j]], r.at[j], sg)

    def drain_gathers(r, sg):
        for j in range(PS):
            pltpu.make_async_copy(e_hbm.at[pl.ds(0, CH)], r.at[j], sg).wait()

    def compute(r):
        def body(pb, _):
            for u in range(8):
                p = pb * 8 + u
                for h in (0, 16):
                    v = (r[0, p, pl.ds(h, 16)] + r[1, p, pl.ds(h, 16)]
                         + r[2, p, pl.ds(h, 16)])
                    r[0, p, pl.ds(h, 16)] = jnp.exp(ALPHA * v - ALPHA)
            return 0

        lax.fori_loop(0, CH // 8, body, 0)

    bs = (b0, b1, b2)
    sis = (si0, si1, si2)
    rs = (r0, r1)
    sgs = (sg0, sg1)
    sss = (ss0, ss1)

    def wait_idx(si):
        pltpu.make_async_copy(g4_hbm.at[cbase], b0, si).wait()

    def wait_scatter(r, b, ss):
        pltpu.make_async_copy(r.at[0], pooled.at[b.at[PS]], ss).wait()

    pltpu.sync_copy(g4_hbm.at[cbase], b0)
    fire_gathers(b0, r0, sg0)
    pltpu.async_copy(g4_hbm.at[cbase + 1], b1, si1)
    pltpu.async_copy(g4_hbm.at[cbase + 2], b2, si2)

    def step(i, _):
        for u in range(6):
            m = 6 * i + u
            drain_gathers(rs[u % 2], sgs[u % 2])

            @pl.when(m > 0)
            def _():
                wait_scatter(rs[(u + 1) % 2], bs[(u + 2) % 3], sss[(u + 1) % 2])

            @pl.when((m > 0) & (m + 2 < CHUNKS))
            def _():
                pltpu.async_copy(g4_hbm.at[cbase + m + 2], bs[(u + 2) % 3],
                                 sis[(u + 2) % 3])

            @pl.when(m + 1 < CHUNKS)
            def _():
                wait_idx(sis[(u + 1) % 3])
                fire_gathers(bs[(u + 1) % 3], rs[(u + 1) % 2], sgs[(u + 1) % 2])

            compute(rs[u % 2])
            pltpu.async_copy(rs[u % 2].at[0], pooled.at[bs[u % 3].at[PS]],
                             sss[u % 2], add=True)

        return 0

    lax.fori_loop(0, CHUNKS // 6, step, 0)
    wait_scatter(rs[(CHUNKS - 1) % 2], bs[(CHUNKS - 1) % 3],
                 sss[(CHUNKS - 1) % 2])
    plsc.subcore_barrier()
    pltpu.sync_copy(pooled.at[pl.ds(s * RPT, RPT)],
                    out_hbm.at[c, pl.ds(s * RPT, RPT)])


_sc_pool = functools.partial(
    pl.kernel,
    mesh=plsc.VectorSubcoreMesh(core_axis_name="c", subcore_axis_name="s"),
    compiler_params=pltpu.CompilerParams(use_tc_tiling_on_sc=False),
    out_type=jax.ShapeDtypeStruct((2, N_POOL, H), jnp.float32),
    scratch_types=[
        pltpu.VMEM((PS + 1, CH), jnp.int32),
        pltpu.VMEM((PS + 1, CH), jnp.int32),
        pltpu.VMEM((PS + 1, CH), jnp.int32),
        pltpu.VMEM((PS, CH, H), jnp.float32),
        pltpu.VMEM((PS, CH, H), jnp.float32),
        pltpu.VMEM_SHARED((N_POOL, H), jnp.float32),
        pltpu.SemaphoreType.DMA,
        pltpu.SemaphoreType.DMA,
        pltpu.SemaphoreType.DMA,
        pltpu.SemaphoreType.DMA,
        pltpu.SemaphoreType.DMA,
        pltpu.SemaphoreType.DMA,
        pltpu.SemaphoreType.DMA,
    ],
)(_sc_body)


def _tc2_body(p_ref, ks_ref, lt_ref, o_ref):
    p = p_ref[0] + p_ref[1]
    cnt = jnp.maximum(ks_ref[...], 1).astype(jnp.float32)
    o_ref[...] = jnp.dot(p / cnt, lt_ref[...], preferred_element_type=jnp.float32)


_tc2 = pl.pallas_call(
    _tc2_body,
    grid=(NB,),
    in_specs=[
        pl.BlockSpec((2, BN, H), lambda i: (0, i, 0)),
        pl.BlockSpec((BN, 1), lambda i: (i, 0)),
        pl.BlockSpec((H, H), lambda i: (0, 0)),
    ],
    out_specs=pl.BlockSpec((BN, H), lambda i: (i, 0)),
    out_shape=jax.ShapeDtypeStruct((N, H), jnp.float32),
)


@jax.jit
def kernel(features, paths_indices, kernel_size, weight):
    e3, lintrans = _tc1(features, weight)
    e_flat = e3.reshape(PS * N, H)

    cum = jnp.cumsum(kernel_size)
    z = jnp.zeros((NP,), jnp.int32).at[cum].add(1, mode='drop')
    seg = jnp.cumsum(z)
    segp = jnp.concatenate([seg, jnp.full((NPP - NP,), N, jnp.int32)])
    pad = jnp.zeros((NPP - NP,), jnp.int32)
    gs = [jnp.concatenate([paths_indices[:, j] + j * N, pad]) for j in range(PS)]
    g4 = jnp.stack(gs + [segp], axis=0).reshape(PS + 1, NPP // CH, CH)
    g4 = g4.transpose(1, 0, 2)

    pooled2 = _sc_pool(e_flat, g4)
    return _tc2(pooled2, kernel_size.reshape(N, 1), lintrans)


# speedup vs baseline: 1.3254x; 1.3203x over previous
import functools

import jax
import jax.numpy as jnp
from jax import lax
from jax.experimental import pallas as pl
from jax.experimental.pallas import tpu as pltpu
from jax.experimental.pallas import tpu_sc as plsc

EPS = 1e-4
ALPHA = 4.0
N = 50000
NP = 800000
PS = 3
D = 128
H = 32

NW = 32
CH = 128
CHUNKS = 196
PPW = CH * CHUNKS
NPP = NW * PPW
N_POOL = 50048
RPT = N_POOL // 16
BN = 5000
NB = N // BN


def _tc1_body(f_ref, w_ref, e_ref, lt_ref):
    w2 = w_ref[...].reshape(PS * D, H)
    colnorm = jnp.maximum(jnp.sqrt(jnp.sum(w2 * w2, axis=0, keepdims=True)), EPS)
    wn = w2 / colnorm
    f = f_ref[...]
    inv = lax.rsqrt(jnp.maximum(jnp.sum(f * f, axis=1, keepdims=True), EPS * EPS))
    fn = f * inv
    wj = wn.reshape(PS, D, H)
    for j in range(PS):
        e_ref[j] = jnp.dot(fn, wj[j], preferred_element_type=jnp.float32) * (1.0 / PS)

    @pl.when(pl.program_id(0) == 0)
    def _():
        gram = jnp.exp(
            ALPHA * (lax.dot_general(wn, wn, (((0,), (0,)), ((), ())),
                                     preferred_element_type=jnp.float32) - 1.0))
        c = jnp.sqrt(jnp.sum(gram * gram))
        eye = jnp.eye(H, dtype=jnp.float32)
        y = gram * (1.0 / c)
        z = eye
        for _ in range(24):
            t = 1.5 * eye - 0.5 * jnp.dot(z, y, preferred_element_type=jnp.float32)
            y = jnp.dot(y, t, preferred_element_type=jnp.float32)
            z = jnp.dot(t, z, preferred_element_type=jnp.float32)
        lt_ref[...] = z * lax.rsqrt(c)


_tc1 = pl.pallas_call(
    _tc1_body,
    grid=(NB,),
    in_specs=[
        pl.BlockSpec((BN, D), lambda i: (i, 0)),
        pl.BlockSpec((PS, D, H), lambda i: (0, 0, 0)),
    ],
    out_specs=[
        pl.BlockSpec((PS, BN, H), lambda i: (0, i, 0)),
        pl.BlockSpec((H, H), lambda i: (0, 0)),
    ],
    out_shape=[
        jax.ShapeDtypeStruct((PS, N, H), jnp.float32),
        jax.ShapeDtypeStruct((H, H), jnp.float32),
    ],
)


def _sc_body(e_hbm, g4_hbm, out_hbm,
             b0, b1, r0, r1, pooled, sem_i, sg0, sg1):
    c = lax.axis_index("c")
    s = lax.axis_index("s")
    wid = s * 2 + c
    cbase = wid * CHUNKS

    def zero_row(p, _):
        for h in (0, 16):
            r0[0, p, pl.ds(h, 16)] = jnp.zeros((16,), jnp.float32)
        return 0

    lax.fori_loop(0, CH, zero_row, 0)

    nfull = RPT // CH
    rem = RPT - nfull * CH

    def zero_pool(i, _):
        pltpu.async_copy(r0.at[0], pooled.at[pl.ds(s * RPT + i * CH, CH)], sg1)
        return 0

    lax.fori_loop(0, nfull, zero_pool, 0)
    pltpu.async_copy(r0.at[0, pl.ds(0, rem)],
                     pooled.at[pl.ds(s * RPT + nfull * CH, rem)], sg0)

    def zero_wait(i, _):
        pltpu.make_async_copy(r0.at[0], pooled.at[pl.ds(s * RPT, CH)], sg1).wait()
        return 0

    lax.fori_loop(0, nfull, zero_wait, 0)
    pltpu.make_async_copy(r0.at[0, pl.ds(0, rem)],
                          pooled.at[pl.ds(s * RPT, rem)], sg0).wait()
    plsc.subcore_barrier()

    def fire_gathers(b, r, sg):
        for j in range(PS):
            pltpu.async_copy(e_hbm.at[b.at[j]], r.at[j], sg)

    def drain_gathers(r, sg):
        for j in range(PS):
            pltpu.make_async_copy(e_hbm.at[pl.ds(0, CH)], r.at[j], sg).wait()

    def compute(r):
        def body(pb, _):
            for u in range(4):
                p = pb * 4 + u
                for h in (0, 16):
                    v = (r[0, p, pl.ds(h, 16)] + r[1, p, pl.ds(h, 16)]
                         + r[2, p, pl.ds(h, 16)])
                    r[0, p, pl.ds(h, 16)] = jnp.exp(ALPHA * v - ALPHA)
            return 0

        lax.fori_loop(0, CH // 4, body, 0)

    pltpu.sync_copy(g4_hbm.at[cbase], b0)
    fire_gathers(b0, r0, sg0)
    pltpu.async_copy(g4_hbm.at[cbase + 1], b1, sem_i)

    def step(i, _):
        for u, bc, rc, sgc, bn, rn, sgn in ((0, b0, r0, sg0, b1, r1, sg1),
                                            (1, b1, r1, sg1, b0, r0, sg0)):
            m = 2 * i + u
            drain_gathers(rc, sgc)

            @pl.when(m + 1 < CHUNKS)
            def _():
                pltpu.make_async_copy(g4_hbm.at[cbase], bn, sem_i).wait()
                fire_gathers(bn, rn, sgn)

            compute(rc)
            pltpu.sync_copy(rc.at[0], pooled.at[bc.at[PS]], add=True)

            @pl.when(m + 2 < CHUNKS)
            def _():
                pltpu.async_copy(g4_hbm.at[cbase + m + 2], bc, sem_i)

        return 0

    lax.fori_loop(0, CHUNKS // 2, step, 0)
    plsc.subcore_barrier()
    pltpu.sync_copy(pooled.at[pl.ds(s * RPT, RPT)],
                    out_hbm.at[c, pl.ds(s * RPT, RPT)])


_sc_pool = functools.partial(
    pl.kernel,
    mesh=plsc.VectorSubcoreMesh(core_axis_name="c", subcore_axis_name="s"),
    compiler_params=pltpu.CompilerParams(use_tc_tiling_on_sc=False),
    out_type=jax.ShapeDtypeStruct((2, N_POOL, H), jnp.float32),
    scratch_types=[
        pltpu.VMEM((PS + 1, CH), jnp.int32),
        pltpu.VMEM((PS + 1, CH), jnp.int32),
        pltpu.VMEM((PS, CH, H), jnp.float32),
        pltpu.VMEM((PS, CH, H), jnp.float32),
        pltpu.VMEM_SHARED((N_POOL, H), jnp.float32),
        pltpu.SemaphoreType.DMA,
        pltpu.SemaphoreType.DMA,
        pltpu.SemaphoreType.DMA,
    ],
)(_sc_body)


def _tc2_body(p_ref, ks_ref, lt_ref, o_ref):
    p = p_ref[0] + p_ref[1]
    cnt = jnp.maximum(ks_ref[...], 1).astype(jnp.float32)
    o_ref[...] = jnp.dot(p / cnt, lt_ref[...], preferred_element_type=jnp.float32)


_tc2 = pl.pallas_call(
    _tc2_body,
    grid=(NB,),
    in_specs=[
        pl.BlockSpec((2, BN, H), lambda i: (0, i, 0)),
        pl.BlockSpec((BN, 1), lambda i: (i, 0)),
        pl.BlockSpec((H, H), lambda i: (0, 0)),
    ],
    out_specs=pl.BlockSpec((BN, H), lambda i: (i, 0)),
    out_shape=jax.ShapeDtypeStruct((N, H), jnp.float32),
)


@jax.jit
def kernel(features, paths_indices, kernel_size, weight):
    e3, lintrans = _tc1(features, weight)
    e_flat = e3.reshape(PS * N, H)

    cum = jnp.cumsum(kernel_size)
    z = jnp.zeros((NP,), jnp.int32).at[cum].add(1, mode='drop')
    seg = jnp.cumsum(z)
    segp = jnp.concatenate([seg, jnp.full((NPP - NP,), N, jnp.int32)])
    pad = jnp.zeros((NPP - NP,), jnp.int32)
    gs = [jnp.concatenate([paths_indices[:, j] + j * N, pad]) for j in range(PS)]
    g4 = jnp.stack(gs + [segp], axis=0).reshape(PS + 1, NPP // CH, CH)
    g4 = g4.transpose(1, 0, 2)

    pooled2 = _sc_pool(e_flat, g4)
    return _tc2(pooled2, kernel_size.reshape(N, 1), lintrans)
